# baseline (speedup 1.0000x reference)
import jax
import jax.numpy as jnp
from jax import lax
from jax.experimental import pallas as pl
from jax.experimental.pallas import tpu as pltpu
from jax.experimental.pallas import tpu_sc as plsc

_NC, _NS = 2, 16
_NW = _NC * _NS
_SUB = 128
_NSUB = 1
_GC = _SUB * _NSUB
_ZR = 48


def _sc_mesh():
    return plsc.VectorSubcoreMesh(core_axis_name="c", subcore_axis_name="s")


_DSUB = 128


def _deg_call(n_pad, n_dgroups):
    gbase, gextra = divmod(n_dgroups, _NW)
    per_tile = n_pad // _NS

    def body(ep_hbm, out_hbm, idx0, idx1, ones_v, zb_v, ds_sp, dd_sp, sem):
        idx = (idx0, idx1)
        c = lax.axis_index("c")
        s = lax.axis_index("s")
        wid = s * _NC + c

        def fill_ones(i, _):
            ones_v[pl.ds(i * 16, 16)] = jnp.full((16,), 1.0, jnp.float32)
            return 0

        lax.fori_loop(0, _DSUB // 16, fill_ones, 0)

        def fill_zero(i, _):
            zb_v[pl.ds(i * 16, 16)] = jnp.zeros((16,), jnp.float32)
            return 0

        lax.fori_loop(0, per_tile // 16, fill_zero, 0)
        row0 = pl.multiple_of(s * per_tile, 8)
        pltpu.sync_copy(zb_v, ds_sp.at[pl.ds(row0, per_tile)])
        pltpu.sync_copy(zb_v, dd_sp.at[pl.ds(row0, per_tile)])
        plsc.subcore_barrier()

        n_my = gbase + jnp.where(wid < gextra, 1, 0)
        pltpu.sync_copy(ep_hbm.at[wid], idx[0])

        def drain2(b):
            pltpu.make_async_copy(ones_v, ds_sp.at[idx[b].at[0]],
                                  sem[b]).wait()
            pltpu.make_async_copy(ones_v, dd_sp.at[idx[b].at[1]],
                                  sem[b]).wait()

        def outer(o, _):
            for b in range(2):
                g = 2 * o + b

                @pl.when(g < n_my)
                def _(g=g, b=b):
                    @pl.when(g >= 2)
                    def _():
                        drain2(b)
                    pltpu.async_copy(ones_v, ds_sp.at[idx[b].at[0]], sem[b],
                                     add=True)
                    pltpu.async_copy(ones_v, dd_sp.at[idx[b].at[1]], sem[b],
                                     add=True)

                    @pl.when(g + 1 < n_my)
                    def _():
                        pltpu.sync_copy(ep_hbm.at[wid + (g + 1) * _NW],
                                        idx[1 - b])
            return 0

        lax.fori_loop(0, (n_my + 1) // 2, outer, 0)
        assert gbase >= 2
        drain2(0)
        drain2(1)
        plsc.subcore_barrier()
        pltpu.sync_copy(ds_sp.at[pl.ds(row0, per_tile)],
                        out_hbm.at[c, 0, pl.ds(row0, per_tile)])
        pltpu.sync_copy(dd_sp.at[pl.ds(row0, per_tile)],
                        out_hbm.at[c, 1, pl.ds(row0, per_tile)])

    def wrapped(ep_hbm, out_hbm, idx0, idx1, ones_v, zb_v, ds_sp, dd_sp,
                sem0, sem1):
        body(ep_hbm, out_hbm, idx0, idx1, ones_v, zb_v, ds_sp, dd_sp,
             (sem0, sem1))

    return pl.kernel(
        wrapped,
        out_type=jax.ShapeDtypeStruct((_NC, 2, n_pad), jnp.float32),
        mesh=_sc_mesh(),
        scratch_types=[
            pltpu.VMEM((2, _DSUB), jnp.int32),
            pltpu.VMEM((2, _DSUB), jnp.int32),
            pltpu.VMEM((_DSUB,), jnp.float32),
            pltpu.VMEM((per_tile,), jnp.float32),
            pltpu.VMEM_SHARED((n_pad,), jnp.float32),
            pltpu.VMEM_SHARED((n_pad,), jnp.float32),
            pltpu.SemaphoreType.DMA,
            pltpu.SemaphoreType.DMA,
        ],
    )


def _agg_call(n, d, n_groups):
    gbase, gextra = divmod(n_groups, _NW)
    rpt = (n // (8 * _NS)) * 8
    rem = n - rpt * _NS
    assert 0 <= rem <= _GC and rem % 8 == 0

    def body(hs_hbm, ep_hbm, out_hbm, idx, rows, zbuf, agg_sp,
             gsem, ssem, isem):
        c = lax.axis_index("c")
        s = lax.axis_index("s")
        wid = s * _NC + c
        n_my = gbase + jnp.where(wid < gextra, 1, 0)
        assert gbase >= 4

        def fire_gathers(g, slot):
            for j in range(_NSUB):
                pltpu.async_copy(hs_hbm.at[idx[slot].at[0, j]],
                                 rows[slot % 2].at[pl.ds(j * _SUB, _SUB)],
                                 gsem[slot % 2])

        pltpu.sync_copy(ep_hbm.at[wid], idx[0])
        pltpu.sync_copy(ep_hbm.at[wid + _NW], idx[1])
        fire_gathers(0, 0)
        fire_gathers(1, 1)
        pltpu.async_copy(ep_hbm.at[wid + 2 * _NW], idx[2], isem[2])
        pltpu.async_copy(ep_hbm.at[wid + 3 * _NW], idx[3], isem[3])

        def zrow(i, _):
            for j in range(d // 16):
                zbuf[i, pl.ds(j * 16, 16)] = jnp.zeros((16,), jnp.float32)
            return 0

        lax.fori_loop(0, _ZR, zrow, 0)
        row0 = pl.multiple_of(s * rpt, 8)
        nfull, tail = divmod(rpt, _ZR)
        for k in range(nfull):
            pltpu.sync_copy(zbuf, agg_sp.at[pl.ds(row0 + k * _ZR, _ZR)])
        if tail:
            pltpu.sync_copy(zbuf.at[pl.ds(0, tail)],
                            agg_sp.at[pl.ds(row0 + nfull * _ZR, tail)])
        if rem:
            @pl.when(s == _NS - 1)
            def _():
                pltpu.sync_copy(zbuf.at[pl.ds(0, rem)],
                                agg_sp.at[pl.ds(rpt * _NS, rem)])
        plsc.subcore_barrier()

        def outer(o, _):
            for b in range(4):
                g = 4 * o + b

                @pl.when(g < n_my)
                def _(g=g, b=b):
                    for j in range(_NSUB):
                        pltpu.make_async_copy(
                            hs_hbm.at[idx[b].at[0, j]],
                            rows[b % 2].at[pl.ds(j * _SUB, _SUB)],
                            gsem[b % 2]).wait()
                    sd = [pltpu.async_copy(
                        rows[b % 2].at[pl.ds(j * _SUB, _SUB)],
                        agg_sp.at[idx[b].at[1, j]], ssem[b % 2], add=True)
                        for j in range(_NSUB)]
                    for dsc in sd:
                        dsc.wait()

                    @pl.when(g + 4 < n_my)
                    def _():
                        pltpu.async_copy(ep_hbm.at[wid + (g + 4) * _NW],
                                         idx[b], isem[b])

                    @pl.when(g + 2 < n_my)
                    def _():
                        slot = (b + 2) % 4
                        pltpu.make_async_copy(
                            ep_hbm.at[wid + (g + 2) * _NW], idx[slot],
                            isem[slot]).wait()
                        fire_gathers(g + 2, slot)
            return 0

        lax.fori_loop(0, (n_my + 3) // 4, outer, 0)
        plsc.subcore_barrier()
        pltpu.sync_copy(agg_sp.at[pl.ds(row0, rpt)],
                        out_hbm.at[c, pl.ds(row0, rpt)])
        if rem:
            @pl.when(s == _NS - 1)
            def _():
                pltpu.sync_copy(agg_sp.at[pl.ds(rpt * _NS, rem)],
                                out_hbm.at[c, pl.ds(rpt * _NS, rem)])

    def wrapped(hs_hbm, ep_hbm, out_hbm,
                i0, i1, i2, i3, r0, r1, zbuf, agg_sp,
                g0, g1, s0, s1, p0, p1, p2, p3):
        body(hs_hbm, ep_hbm, out_hbm, (i0, i1, i2, i3), (r0, r1), zbuf,
             agg_sp, (g0, g1), (s0, s1), (p0, p1, p2, p3))

    ivm = pltpu.VMEM((2, _NSUB, _SUB), jnp.int32)
    return pl.kernel(
        wrapped,
        out_type=jax.ShapeDtypeStruct((_NC, n, d), jnp.float32),
        mesh=_sc_mesh(),
        scratch_types=[
            ivm, ivm, ivm, ivm,
            pltpu.VMEM((_GC, d), jnp.float32),
            pltpu.VMEM((_GC, d), jnp.float32),
            pltpu.VMEM((_ZR, d), jnp.float32),
            pltpu.VMEM_SHARED((n, d), jnp.float32),
        ] + [pltpu.SemaphoreType.DMA] * 8,
    )


def _prep_call(n, d, blk):
    g = (n + blk - 1) // blk

    def body(x_ref, dp_ref, hs_ref):
        dpt = jnp.transpose(dp_ref[...])
        deg_out = dpt[:, 0:1] + dpt[:, 2:3]
        hs_ref[...] = x_ref[...] * lax.rsqrt(jnp.maximum(deg_out, 1.0))

    return pl.pallas_call(
        body,
        grid=(g,),
        in_specs=[pl.BlockSpec((blk, d), lambda i: (i, 0)),
                  pl.BlockSpec((4, blk), lambda i: (0, i))],
        out_specs=pl.BlockSpec((blk, d), lambda i: (i, 0)),
        out_shape=jax.ShapeDtypeStruct((n, d), jnp.float32),
    )


def _layer_call(n, d, blk, last):
    g = (n + blk - 1) // blk

    def body(p_ref, dp_ref, w_ref, b_ref, o_ref):
        agg = p_ref[0] + p_ref[1]
        dpt = jnp.transpose(dp_ref[...])
        deg_in = dpt[:, 1:2] + dpt[:, 3:4]
        agg = agg * lax.rsqrt(jnp.maximum(deg_in, 1.0))
        y = jnp.dot(agg, w_ref[...], preferred_element_type=jnp.float32)
        y = y + b_ref[...]
        if not last:
            deg_out = dpt[:, 0:1] + dpt[:, 2:3]
            y = y * lax.rsqrt(jnp.maximum(deg_out, 1.0))
        o_ref[...] = y

    return pl.pallas_call(
        body,
        grid=(g,),
        in_specs=[pl.BlockSpec((_NC, blk, d), lambda i: (0, i, 0)),
                  pl.BlockSpec((4, blk), lambda i: (0, i)),
                  pl.BlockSpec((d, d), lambda i: (0, 0)),
                  pl.BlockSpec((1, d), lambda i: (0, 0))],
        out_specs=pl.BlockSpec((blk, d), lambda i: (i, 0)),
        out_shape=jax.ShapeDtypeStruct((n, d), jnp.float32),
    )


def kernel(node_embeddings, edge_matrix, edge_labels_or_dummynode_ids,
           W0, b0, W1, b1, W2, b2, W3, b3):
    del edge_labels_or_dummynode_ids
    x = node_embeddings
    n, d = x.shape
    e = edge_matrix.shape[1]
    assert e % _GC == 0 and e % _DSUB == 0 and n % _NS == 0 and d % 16 == 0
    n_groups = e // _GC
    blk = 512
    n_pad = ((n + blk - 1) // blk) * blk

    ep = edge_matrix.reshape(2, n_groups, _NSUB, _SUB).transpose(1, 0, 2, 3)
    assert _GC == _DSUB
    epd = ep.reshape(n_groups, 2, _DSUB)

    degp = _deg_call(n_pad, e // _DSUB)(epd)
    dp = degp.reshape(2 * _NC, n_pad)

    hs = _prep_call(n, d, blk)(x, dp)
    for i, (w, b) in enumerate(((W0, b0), (W1, b1), (W2, b2), (W3, b3))):
        partial = _agg_call(n, d, n_groups)(hs, ep)
        hs = _layer_call(n, d, blk, i == 3)(partial, dp, w, b.reshape(1, d))
    return hs


# speedup vs baseline: 1.0553x; 1.0553x over previous
import jax
import jax.numpy as jnp
from jax import lax
from jax.experimental import pallas as pl
from jax.experimental.pallas import tpu as pltpu
from jax.experimental.pallas import tpu_sc as plsc

_NC, _NS = 2, 16
_NW = _NC * _NS
_SUB = 128
_NSUB = 1
_GC = _SUB * _NSUB
_ZR = 48


def _sc_mesh():
    return plsc.VectorSubcoreMesh(core_axis_name="c", subcore_axis_name="s")


_DSUB = 128


def _deg_call(n_pad, n_dgroups):
    gbase, gextra = divmod(n_dgroups, _NW)
    per_tile = n_pad // _NS

    def body(ep_hbm, out_hbm, idx0, idx1, ones_v, zb_v, ds_sp, dd_sp, sem):
        idx = (idx0, idx1)
        c = lax.axis_index("c")
        s = lax.axis_index("s")
        wid = s * _NC + c

        def fill_ones(i, _):
            ones_v[pl.ds(i * 16, 16)] = jnp.full((16,), 1.0, jnp.float32)
            return 0

        lax.fori_loop(0, _DSUB // 16, fill_ones, 0)

        def fill_zero(i, _):
            zb_v[pl.ds(i * 16, 16)] = jnp.zeros((16,), jnp.float32)
            return 0

        lax.fori_loop(0, per_tile // 16, fill_zero, 0)
        row0 = pl.multiple_of(s * per_tile, 8)
        pltpu.sync_copy(zb_v, ds_sp.at[pl.ds(row0, per_tile)])
        pltpu.sync_copy(zb_v, dd_sp.at[pl.ds(row0, per_tile)])
        plsc.subcore_barrier()

        n_my = gbase + jnp.where(wid < gextra, 1, 0)
        pltpu.sync_copy(ep_hbm.at[wid], idx[0])

        def drain2(b):
            pltpu.make_async_copy(ones_v, ds_sp.at[idx[b].at[0]],
                                  sem[b]).wait()
            pltpu.make_async_copy(ones_v, dd_sp.at[idx[b].at[1]],
                                  sem[b]).wait()

        def outer(o, _):
            for b in range(2):
                g = 2 * o + b

                @pl.when(g < n_my)
                def _(g=g, b=b):
                    @pl.when(g >= 2)
                    def _():
                        drain2(b)
                    pltpu.async_copy(ones_v, ds_sp.at[idx[b].at[0]], sem[b],
                                     add=True)
                    pltpu.async_copy(ones_v, dd_sp.at[idx[b].at[1]], sem[b],
                                     add=True)

                    @pl.when(g + 1 < n_my)
                    def _():
                        pltpu.sync_copy(ep_hbm.at[wid + (g + 1) * _NW],
                                        idx[1 - b])
            return 0

        lax.fori_loop(0, (n_my + 1) // 2, outer, 0)
        assert gbase >= 2
        drain2(0)
        drain2(1)
        plsc.subcore_barrier()
        pltpu.sync_copy(ds_sp.at[pl.ds(row0, per_tile)],
                        out_hbm.at[c, 0, pl.ds(row0, per_tile)])
        pltpu.sync_copy(dd_sp.at[pl.ds(row0, per_tile)],
                        out_hbm.at[c, 1, pl.ds(row0, per_tile)])

    def wrapped(ep_hbm, out_hbm, idx0, idx1, ones_v, zb_v, ds_sp, dd_sp,
                sem0, sem1):
        body(ep_hbm, out_hbm, idx0, idx1, ones_v, zb_v, ds_sp, dd_sp,
             (sem0, sem1))

    return pl.kernel(
        wrapped,
        out_type=jax.ShapeDtypeStruct((_NC, 2, n_pad), jnp.float32),
        mesh=_sc_mesh(),
        scratch_types=[
            pltpu.VMEM((2, _DSUB), jnp.int32),
            pltpu.VMEM((2, _DSUB), jnp.int32),
            pltpu.VMEM((_DSUB,), jnp.float32),
            pltpu.VMEM((per_tile,), jnp.float32),
            pltpu.VMEM_SHARED((n_pad,), jnp.float32),
            pltpu.VMEM_SHARED((n_pad,), jnp.float32),
            pltpu.SemaphoreType.DMA,
            pltpu.SemaphoreType.DMA,
        ],
    )


def _agg_call(n, d, n_groups):
    gbase, gextra = divmod(n_groups, _NW)
    rpt = (n // (8 * _NS)) * 8
    rem = n - rpt * _NS
    assert 0 <= rem <= _GC and rem % 8 == 0

    def body(hs_hbm, ep_hbm, out_hbm, idx, rows, zbuf, agg_sp,
             gsem, ssem, isem):
        c = lax.axis_index("c")
        s = lax.axis_index("s")
        wid = s * _NC + c
        n_my = gbase + jnp.where(wid < gextra, 1, 0)
        assert gbase >= 4

        def fire_gathers(g, slot):
            for j in range(_NSUB):
                pltpu.async_copy(hs_hbm.at[idx[slot].at[0, j]],
                                 rows[slot % 2].at[pl.ds(j * _SUB, _SUB)],
                                 gsem[slot % 2])

        pltpu.sync_copy(ep_hbm.at[wid], idx[0])
        pltpu.sync_copy(ep_hbm.at[wid + _NW], idx[1])
        fire_gathers(0, 0)
        fire_gathers(1, 1)
        pltpu.async_copy(ep_hbm.at[wid + 2 * _NW], idx[2], isem[2])
        pltpu.async_copy(ep_hbm.at[wid + 3 * _NW], idx[3], isem[3])

        def zrow(i, _):
            for j in range(d // 16):
                zbuf[i, pl.ds(j * 16, 16)] = jnp.zeros((16,), jnp.float32)
            return 0

        lax.fori_loop(0, _ZR, zrow, 0)
        row0 = pl.multiple_of(s * rpt, 8)
        nfull, tail = divmod(rpt, _ZR)
        for k in range(nfull):
            pltpu.sync_copy(zbuf, agg_sp.at[pl.ds(row0 + k * _ZR, _ZR)])
        if tail:
            pltpu.sync_copy(zbuf.at[pl.ds(0, tail)],
                            agg_sp.at[pl.ds(row0 + nfull * _ZR, tail)])
        if rem:
            @pl.when(s == _NS - 1)
            def _():
                pltpu.sync_copy(zbuf.at[pl.ds(0, rem)],
                                agg_sp.at[pl.ds(rpt * _NS, rem)])
        plsc.subcore_barrier()

        def outer(o, _):
            for b in range(4):
                g = 4 * o + b

                @pl.when(g < n_my)
                def _(g=g, b=b):
                    for j in range(_NSUB):
                        pltpu.make_async_copy(
                            hs_hbm.at[idx[b].at[0, j]],
                            rows[b % 2].at[pl.ds(j * _SUB, _SUB)],
                            gsem[b % 2]).wait()
                    sd = [pltpu.async_copy(
                        rows[b % 2].at[pl.ds(j * _SUB, _SUB)],
                        agg_sp.at[idx[b].at[1, j]], ssem[b % 2], add=True)
                        for j in range(_NSUB)]
                    for dsc in sd:
                        dsc.wait()

                    @pl.when(g + 4 < n_my)
                    def _():
                        pltpu.async_copy(ep_hbm.at[wid + (g + 4) * _NW],
                                         idx[b], isem[b])

                    @pl.when(g + 2 < n_my)
                    def _():
                        slot = (b + 2) % 4
                        pltpu.make_async_copy(
                            ep_hbm.at[wid + (g + 2) * _NW], idx[slot],
                            isem[slot]).wait()
                        fire_gathers(g + 2, slot)
            return 0

        lax.fori_loop(0, (n_my + 3) // 4, outer, 0)
        plsc.subcore_barrier()
        pltpu.sync_copy(agg_sp.at[pl.ds(row0, rpt)],
                        out_hbm.at[c, pl.ds(row0, rpt)])
        if rem:
            @pl.when(s == _NS - 1)
            def _():
                pltpu.sync_copy(agg_sp.at[pl.ds(rpt * _NS, rem)],
                                out_hbm.at[c, pl.ds(rpt * _NS, rem)])

    def wrapped(hs_hbm, ep_hbm, out_hbm,
                i0, i1, i2, i3, r0, r1, zbuf, agg_sp,
                g0, g1, s0, s1, p0, p1, p2, p3):
        body(hs_hbm, ep_hbm, out_hbm, (i0, i1, i2, i3), (r0, r1), zbuf,
             agg_sp, (g0, g1), (s0, s1), (p0, p1, p2, p3))

    ivm = pltpu.VMEM((2, _NSUB, _SUB), jnp.int32)
    return pl.kernel(
        wrapped,
        out_type=jax.ShapeDtypeStruct((_NC, n, d), jnp.float32),
        mesh=_sc_mesh(),
        scratch_types=[
            ivm, ivm, ivm, ivm,
            pltpu.VMEM((_GC, d), jnp.float32),
            pltpu.VMEM((_GC, d), jnp.float32),
            pltpu.VMEM((_ZR, d), jnp.float32),
            pltpu.VMEM_SHARED((n, d), jnp.float32),
        ] + [pltpu.SemaphoreType.DMA] * 8,
    )


def _prep_call(n, d, blk):
    g = (n + blk - 1) // blk

    def body(x_ref, dp_ref, hs_ref):
        dpt = jnp.transpose(dp_ref[...])
        deg_out = dpt[:, 0:1] + dpt[:, 2:3]
        hs_ref[...] = x_ref[...] * lax.rsqrt(jnp.maximum(deg_out, 1.0))

    return pl.pallas_call(
        body,
        grid=(g,),
        in_specs=[pl.BlockSpec((blk, d), lambda i: (i, 0)),
                  pl.BlockSpec((4, blk), lambda i: (0, i))],
        out_specs=pl.BlockSpec((blk, d), lambda i: (i, 0)),
        out_shape=jax.ShapeDtypeStruct((n, d), jnp.float32),
    )


def _layer_call(n, d, blk, last):
    g = (n + blk - 1) // blk

    def body(p_ref, dp_ref, w_ref, b_ref, o_ref):
        agg = p_ref[0] + p_ref[1]
        dpt = jnp.transpose(dp_ref[...])
        deg_in = dpt[:, 1:2] + dpt[:, 3:4]
        agg = agg * lax.rsqrt(jnp.maximum(deg_in, 1.0))
        y = jnp.dot(agg, w_ref[...], preferred_element_type=jnp.float32)
        y = y + b_ref[...]
        if not last:
            deg_out = dpt[:, 0:1] + dpt[:, 2:3]
            y = y * lax.rsqrt(jnp.maximum(deg_out, 1.0))
        o_ref[...] = y

    return pl.pallas_call(
        body,
        grid=(g,),
        in_specs=[pl.BlockSpec((_NC, blk, d), lambda i: (0, i, 0)),
                  pl.BlockSpec((4, blk), lambda i: (0, i)),
                  pl.BlockSpec((d, d), lambda i: (0, 0)),
                  pl.BlockSpec((1, d), lambda i: (0, 0))],
        out_specs=pl.BlockSpec((blk, d), lambda i: (i, 0)),
        out_shape=jax.ShapeDtypeStruct((n, d), jnp.float32),
    )


def kernel(node_embeddings, edge_matrix, edge_labels_or_dummynode_ids,
           W0, b0, W1, b1, W2, b2, W3, b3):
    del edge_labels_or_dummynode_ids
    x = node_embeddings
    n, d = x.shape
    e = edge_matrix.shape[1]
    assert e % _GC == 0 and e % _DSUB == 0 and n % _NS == 0 and d % 16 == 0
    n_groups = e // _GC
    blk = 1024
    n_pad = ((n + blk - 1) // blk) * blk

    ep = edge_matrix.reshape(2, n_groups, _NSUB, _SUB).transpose(1, 0, 2, 3)
    assert _GC == _DSUB
    epd = ep.reshape(n_groups, 2, _DSUB)

    degp = _deg_call(n_pad, e // _DSUB)(epd)
    dp = degp.reshape(2 * _NC, n_pad)

    hs = _prep_call(n, d, blk)(x, dp)
    for i, (w, b) in enumerate(((W0, b0), (W1, b1), (W2, b2), (W3, b3))):
        partial = _agg_call(n, d, n_groups)(hs, ep)
        hs = _layer_call(n, d, blk, i == 3)(partial, dp, w, b.reshape(1, d))
    return hs


# speedup vs baseline: 1.0781x; 1.0217x over previous
import jax
import jax.numpy as jnp
from jax import lax
from jax.experimental import pallas as pl
from jax.experimental.pallas import tpu as pltpu
from jax.experimental.pallas import tpu_sc as plsc

_NC, _NS = 2, 16
_NW = _NC * _NS
_SUB = 128
_NSUB = 1
_GC = _SUB * _NSUB
_ZR = 48


def _sc_mesh():
    return plsc.VectorSubcoreMesh(core_axis_name="c", subcore_axis_name="s")


_DSUB = 128


def _deg_call(n_pad, n_dgroups):
    gbase, gextra = divmod(n_dgroups, _NW)
    per_tile = n_pad // _NS

    def body(ep_hbm, out_hbm, idx0, idx1, ones_v, zb_v, ds_sp, dd_sp, sem):
        idx = (idx0, idx1)
        c = lax.axis_index("c")
        s = lax.axis_index("s")
        wid = s * _NC + c

        def fill_ones(i, _):
            ones_v[pl.ds(i * 16, 16)] = jnp.full((16,), 1.0, jnp.float32)
            return 0

        lax.fori_loop(0, _DSUB // 16, fill_ones, 0)

        def fill_zero(i, _):
            zb_v[pl.ds(i * 16, 16)] = jnp.zeros((16,), jnp.float32)
            return 0

        lax.fori_loop(0, per_tile // 16, fill_zero, 0)
        row0 = pl.multiple_of(s * per_tile, 8)
        pltpu.sync_copy(zb_v, ds_sp.at[pl.ds(row0, per_tile)])
        pltpu.sync_copy(zb_v, dd_sp.at[pl.ds(row0, per_tile)])
        plsc.subcore_barrier()

        n_my = gbase + jnp.where(wid < gextra, 1, 0)
        pltpu.sync_copy(ep_hbm.at[wid], idx[0])

        def drain2(b):
            pltpu.make_async_copy(ones_v, ds_sp.at[idx[b].at[0]],
                                  sem[b]).wait()
            pltpu.make_async_copy(ones_v, dd_sp.at[idx[b].at[1]],
                                  sem[b]).wait()

        def outer(o, _):
            for b in range(2):
                g = 2 * o + b

                @pl.when(g < n_my)
                def _(g=g, b=b):
                    @pl.when(g >= 2)
                    def _():
                        drain2(b)
                    pltpu.async_copy(ones_v, ds_sp.at[idx[b].at[0]], sem[b],
                                     add=True)
                    pltpu.async_copy(ones_v, dd_sp.at[idx[b].at[1]], sem[b],
                                     add=True)

                    @pl.when(g + 1 < n_my)
                    def _():
                        pltpu.sync_copy(ep_hbm.at[wid + (g + 1) * _NW],
                                        idx[1 - b])
            return 0

        lax.fori_loop(0, (n_my + 1) // 2, outer, 0)
        assert gbase >= 2
        drain2(0)
        drain2(1)
        plsc.subcore_barrier()
        pltpu.sync_copy(ds_sp.at[pl.ds(row0, per_tile)],
                        out_hbm.at[c, 0, pl.ds(row0, per_tile)])
        pltpu.sync_copy(dd_sp.at[pl.ds(row0, per_tile)],
                        out_hbm.at[c, 1, pl.ds(row0, per_tile)])

    def wrapped(ep_hbm, out_hbm, idx0, idx1, ones_v, zb_v, ds_sp, dd_sp,
                sem0, sem1):
        body(ep_hbm, out_hbm, idx0, idx1, ones_v, zb_v, ds_sp, dd_sp,
             (sem0, sem1))

    return pl.kernel(
        wrapped,
        out_type=jax.ShapeDtypeStruct((_NC, 2, n_pad), jnp.float32),
        mesh=_sc_mesh(),
        scratch_types=[
            pltpu.VMEM((2, _DSUB), jnp.int32),
            pltpu.VMEM((2, _DSUB), jnp.int32),
            pltpu.VMEM((_DSUB,), jnp.float32),
            pltpu.VMEM((per_tile,), jnp.float32),
            pltpu.VMEM_SHARED((n_pad,), jnp.float32),
            pltpu.VMEM_SHARED((n_pad,), jnp.float32),
            pltpu.SemaphoreType.DMA,
            pltpu.SemaphoreType.DMA,
        ],
    )


def _agg_call(n, d, n_groups):
    gbase, gextra = divmod(n_groups, _NW)
    rpt = (n // (8 * _NS)) * 8
    rem = n - rpt * _NS
    assert 0 <= rem <= _GC and rem % 8 == 0

    def body(hs_hbm, ep_hbm, out_hbm, idx, rows, zbuf, agg_sp,
             gsem, ssem, isem):
        c = lax.axis_index("c")
        s = lax.axis_index("s")
        wid = s * _NC + c
        n_my = gbase + jnp.where(wid < gextra, 1, 0)
        assert gbase >= 4

        def fire_gathers(g, slot):
            for j in range(_NSUB):
                pltpu.async_copy(hs_hbm.at[idx[slot].at[0, j]],
                                 rows[slot % 2].at[pl.ds(j * _SUB, _SUB)],
                                 gsem[slot % 2])

        pltpu.sync_copy(ep_hbm.at[wid], idx[0])
        pltpu.sync_copy(ep_hbm.at[wid + _NW], idx[1])
        fire_gathers(0, 0)
        fire_gathers(1, 1)
        pltpu.async_copy(ep_hbm.at[wid + 2 * _NW], idx[2], isem[2])
        pltpu.async_copy(ep_hbm.at[wid + 3 * _NW], idx[3], isem[3])

        def zrow(i, _):
            for j in range(d // 16):
                zbuf[i, pl.ds(j * 16, 16)] = jnp.zeros((16,), jnp.float32)
            return 0

        lax.fori_loop(0, _ZR, zrow, 0)
        row0 = pl.multiple_of(s * rpt, 8)
        nfull, tail = divmod(rpt, _ZR)
        for k in range(nfull):
            pltpu.sync_copy(zbuf, agg_sp.at[pl.ds(row0 + k * _ZR, _ZR)])
        if tail:
            pltpu.sync_copy(zbuf.at[pl.ds(0, tail)],
                            agg_sp.at[pl.ds(row0 + nfull * _ZR, tail)])
        if rem:
            @pl.when(s == _NS - 1)
            def _():
                pltpu.sync_copy(zbuf.at[pl.ds(0, rem)],
                                agg_sp.at[pl.ds(rpt * _NS, rem)])
        plsc.subcore_barrier()

        def outer(o, _):
            for b in range(4):
                g = 4 * o + b

                @pl.when(g < n_my)
                def _(g=g, b=b):
                    for j in range(_NSUB):
                        pltpu.make_async_copy(
                            hs_hbm.at[idx[b].at[0, j]],
                            rows[b % 2].at[pl.ds(j * _SUB, _SUB)],
                            gsem[b % 2]).wait()
                    sd = [pltpu.async_copy(
                        rows[b % 2].at[pl.ds(j * _SUB, _SUB)],
                        agg_sp.at[idx[b].at[1, j]], ssem[b % 2], add=True)
                        for j in range(_NSUB)]
                    for dsc in sd:
                        dsc.wait()

                    @pl.when(g + 4 < n_my)
                    def _():
                        pltpu.async_copy(ep_hbm.at[wid + (g + 4) * _NW],
                                         idx[b], isem[b])

                    @pl.when(g + 2 < n_my)
                    def _():
                        slot = (b + 2) % 4
                        pltpu.make_async_copy(
                            ep_hbm.at[wid + (g + 2) * _NW], idx[slot],
                            isem[slot]).wait()
                        fire_gathers(g + 2, slot)
            return 0

        lax.fori_loop(0, (n_my + 3) // 4, outer, 0)
        plsc.subcore_barrier()
        pltpu.sync_copy(agg_sp.at[pl.ds(row0, rpt)],
                        out_hbm.at[c, pl.ds(row0, rpt)])
        if rem:
            @pl.when(s == _NS - 1)
            def _():
                pltpu.sync_copy(agg_sp.at[pl.ds(rpt * _NS, rem)],
                                out_hbm.at[c, pl.ds(rpt * _NS, rem)])

    def wrapped(hs_hbm, ep_hbm, out_hbm,
                i0, i1, i2, i3, r0, r1, zbuf, agg_sp,
                g0, g1, s0, s1, p0, p1, p2, p3):
        body(hs_hbm, ep_hbm, out_hbm, (i0, i1, i2, i3), (r0, r1), zbuf,
             agg_sp, (g0, g1), (s0, s1), (p0, p1, p2, p3))

    ivm = pltpu.VMEM((2, _NSUB, _SUB), jnp.int32)
    return pl.kernel(
        wrapped,
        out_type=jax.ShapeDtypeStruct((_NC, n, d), jnp.float32),
        mesh=_sc_mesh(),
        scratch_types=[
            ivm, ivm, ivm, ivm,
            pltpu.VMEM((_GC, d), jnp.float32),
            pltpu.VMEM((_GC, d), jnp.float32),
            pltpu.VMEM((_ZR, d), jnp.float32),
            pltpu.VMEM_SHARED((n, d), jnp.float32),
        ] + [pltpu.SemaphoreType.DMA] * 8,
    )


def _prep_call(n, d, blk):
    g = (n + blk - 1) // blk

    def body(x_ref, dp_ref, hs_ref):
        dpt = jnp.transpose(dp_ref[...])
        deg_out = dpt[:, 0:1] + dpt[:, 2:3]
        hs_ref[...] = x_ref[...] * lax.rsqrt(jnp.maximum(deg_out, 1.0))

    return pl.pallas_call(
        body,
        grid=(g,),
        in_specs=[pl.BlockSpec((blk, d), lambda i: (i, 0)),
                  pl.BlockSpec((4, blk), lambda i: (0, i))],
        out_specs=pl.BlockSpec((blk, d), lambda i: (i, 0)),
        out_shape=jax.ShapeDtypeStruct((n, d), jnp.float32),
    )


def _layer_call(n, d, blk, last):
    g = (n + blk - 1) // blk

    def body(p_ref, dp_ref, w_ref, b_ref, o_ref):
        agg = p_ref[0] + p_ref[1]
        dpt = jnp.transpose(dp_ref[...])
        deg_in = dpt[:, 1:2] + dpt[:, 3:4]
        agg = agg * lax.rsqrt(jnp.maximum(deg_in, 1.0))
        y = jnp.dot(agg, w_ref[...], preferred_element_type=jnp.float32)
        y = y + b_ref[...]
        if not last:
            deg_out = dpt[:, 0:1] + dpt[:, 2:3]
            y = y * lax.rsqrt(jnp.maximum(deg_out, 1.0))
        o_ref[...] = y

    return pl.pallas_call(
        body,
        grid=(g,),
        in_specs=[pl.BlockSpec((_NC, blk, d), lambda i: (0, i, 0)),
                  pl.BlockSpec((4, blk), lambda i: (0, i)),
                  pl.BlockSpec((d, d), lambda i: (0, 0)),
                  pl.BlockSpec((1, d), lambda i: (0, 0))],
        out_specs=pl.BlockSpec((blk, d), lambda i: (i, 0)),
        out_shape=jax.ShapeDtypeStruct((n, d), jnp.float32),
    )


def kernel(node_embeddings, edge_matrix, edge_labels_or_dummynode_ids,
           W0, b0, W1, b1, W2, b2, W3, b3):
    del edge_labels_or_dummynode_ids
    x = node_embeddings
    n, d = x.shape
    e = edge_matrix.shape[1]
    assert e % _GC == 0 and e % _DSUB == 0 and n % _NS == 0 and d % 16 == 0
    n_groups = e // _GC
    blk = 2048
    n_pad = ((n + blk - 1) // blk) * blk

    ep = edge_matrix.reshape(2, n_groups, _NSUB, _SUB).transpose(1, 0, 2, 3)
    assert _GC == _DSUB
    epd = ep.reshape(n_groups, 2, _DSUB)

    degp = _deg_call(n_pad, e // _DSUB)(epd)
    dp = degp.reshape(2 * _NC, n_pad)

    hs = _prep_call(n, d, blk)(x, dp)
    for i, (w, b) in enumerate(((W0, b0), (W1, b1), (W2, b2), (W3, b3))):
        partial = _agg_call(n, d, n_groups)(hs, ep)
        hs = _layer_call(n, d, blk, i == 3)(partial, dp, w, b.reshape(1, d))
    return hs


# speedup vs baseline: 1.1176x; 1.0366x over previous
import jax
import jax.numpy as jnp
from jax import lax
from jax.experimental import pallas as pl
from jax.experimental.pallas import tpu as pltpu
from jax.experimental.pallas import tpu_sc as plsc

_NC, _NS = 2, 16
_NW = _NC * _NS
_SUB = 80
_NSUB = 1
_GC = _SUB * _NSUB


def _sc_mesh():
    return plsc.VectorSubcoreMesh(core_axis_name="c", subcore_axis_name="s")


_DSUB = 128


def _deg_call(n_pad, n_dgroups):
    gbase, gextra = divmod(n_dgroups, _NW)
    per_tile = n_pad // _NS

    def body(ep_hbm, out_hbm, idx0, idx1, ones_v, zb_v, ds_sp, dd_sp, sem):
        idx = (idx0, idx1)
        c = lax.axis_index("c")
        s = lax.axis_index("s")
        wid = s * _NC + c

        def fill_ones(i, _):
            ones_v[pl.ds(i * 16, 16)] = jnp.full((16,), 1.0, jnp.float32)
            return 0

        lax.fori_loop(0, _DSUB // 16, fill_ones, 0)

        def fill_zero(i, _):
            zb_v[pl.ds(i * 16, 16)] = jnp.zeros((16,), jnp.float32)
            return 0

        lax.fori_loop(0, per_tile // 16, fill_zero, 0)
        row0 = pl.multiple_of(s * per_tile, 8)
        pltpu.sync_copy(zb_v, ds_sp.at[pl.ds(row0, per_tile)])
        pltpu.sync_copy(zb_v, dd_sp.at[pl.ds(row0, per_tile)])
        plsc.subcore_barrier()

        n_my = gbase + jnp.where(wid < gextra, 1, 0)
        pltpu.sync_copy(ep_hbm.at[wid], idx[0])

        def drain2(b):
            pltpu.make_async_copy(ones_v, ds_sp.at[idx[b].at[0]],
                                  sem[b]).wait()
            pltpu.make_async_copy(ones_v, dd_sp.at[idx[b].at[1]],
                                  sem[b]).wait()

        def outer(o, _):
            for b in range(2):
                g = 2 * o + b

                @pl.when(g < n_my)
                def _(g=g, b=b):
                    @pl.when(g >= 2)
                    def _():
                        drain2(b)
                    pltpu.async_copy(ones_v, ds_sp.at[idx[b].at[0]], sem[b],
                                     add=True)
                    pltpu.async_copy(ones_v, dd_sp.at[idx[b].at[1]], sem[b],
                                     add=True)

                    @pl.when(g + 1 < n_my)
                    def _():
                        pltpu.sync_copy(ep_hbm.at[wid + (g + 1) * _NW],
                                        idx[1 - b])
            return 0

        lax.fori_loop(0, (n_my + 1) // 2, outer, 0)
        assert gbase >= 2
        drain2(0)
        drain2(1)
        plsc.subcore_barrier()
        pltpu.sync_copy(ds_sp.at[pl.ds(row0, per_tile)],
                        out_hbm.at[c, 0, pl.ds(row0, per_tile)])
        pltpu.sync_copy(dd_sp.at[pl.ds(row0, per_tile)],
                        out_hbm.at[c, 1, pl.ds(row0, per_tile)])

    def wrapped(ep_hbm, out_hbm, idx0, idx1, ones_v, zb_v, ds_sp, dd_sp,
                sem0, sem1):
        body(ep_hbm, out_hbm, idx0, idx1, ones_v, zb_v, ds_sp, dd_sp,
             (sem0, sem1))

    return pl.kernel(
        wrapped,
        out_type=jax.ShapeDtypeStruct((_NC, 2, n_pad), jnp.float32),
        mesh=_sc_mesh(),
        scratch_types=[
            pltpu.VMEM((2, _DSUB), jnp.int32),
            pltpu.VMEM((2, _DSUB), jnp.int32),
            pltpu.VMEM((_DSUB,), jnp.float32),
            pltpu.VMEM((per_tile,), jnp.float32),
            pltpu.VMEM_SHARED((n_pad,), jnp.float32),
            pltpu.VMEM_SHARED((n_pad,), jnp.float32),
            pltpu.SemaphoreType.DMA,
            pltpu.SemaphoreType.DMA,
        ],
    )


def _agg_call(n, d, n_groups):
    gbase, gextra = divmod(n_groups, _NW)
    rpt = (n // (8 * _NS)) * 8
    rem = n - rpt * _NS
    assert 0 <= rem <= _GC and rem % 8 == 0

    def body(hs_hbm, ep_hbm, out_hbm, idx, rows, agg_sp, gsem, ssem, isem):
        c = lax.axis_index("c")
        s = lax.axis_index("s")
        wid = s * _NC + c
        n_my = gbase + jnp.where(wid < gextra, 1, 0)
        assert gbase >= 6

        def gather(g, slot):
            return pltpu.async_copy(hs_hbm.at[idx[slot % 6].at[0, 0]],
                                    rows[slot % 3], gsem[slot % 3])

        def scatter(g, slot):
            return pltpu.async_copy(rows[slot % 3],
                                    agg_sp.at[idx[slot % 6].at[1, 0]],
                                    ssem[slot % 3], add=True)

        pltpu.sync_copy(ep_hbm.at[wid], idx[0])
        pltpu.sync_copy(ep_hbm.at[wid + _NW], idx[1])
        gather(1, 1)
        pltpu.async_copy(ep_hbm.at[wid + 2 * _NW], idx[2], isem[2])
        pltpu.async_copy(ep_hbm.at[wid + 3 * _NW], idx[3], isem[3])

        def zrow(i, _):
            for j in range(d // 16):
                rows[0][i, pl.ds(j * 16, 16)] = jnp.zeros((16,), jnp.float32)
            return 0

        lax.fori_loop(0, _GC, zrow, 0)
        row0 = pl.multiple_of(s * rpt, 8)
        nfull, tail = divmod(rpt, _GC)
        for k in range(nfull):
            pltpu.sync_copy(rows[0], agg_sp.at[pl.ds(row0 + k * _GC, _GC)])
        if tail:
            pltpu.sync_copy(rows[0].at[pl.ds(0, tail)],
                            agg_sp.at[pl.ds(row0 + nfull * _GC, tail)])
        if rem:
            @pl.when(s == _NS - 1)
            def _():
                pltpu.sync_copy(rows[0].at[pl.ds(0, rem)],
                                agg_sp.at[pl.ds(rpt * _NS, rem)])
        gather(0, 0)
        plsc.subcore_barrier()

        def outer(o, _):
            for b in range(6):
                g = 6 * o + b

                @pl.when(g < n_my)
                def _(g=g, b=b):
                    @pl.when(g >= 2)
                    def _():
                        pltpu.make_async_copy(
                            rows[(b + 1) % 3],
                            agg_sp.at[idx[(b + 1) % 6].at[1, 0]],
                            ssem[(b + 1) % 3]).wait()

                    @pl.when((g >= 1) & (g + 1 < n_my))
                    def _():
                        pltpu.make_async_copy(
                            ep_hbm.at[wid + (g + 1) * _NW],
                            idx[(b + 1) % 6],
                            isem[(b + 1) % 6]).wait()
                        gather(g + 1, b + 1)
                    pltpu.make_async_copy(hs_hbm.at[idx[b].at[0, 0]],
                                          rows[b % 3], gsem[b % 3]).wait()
                    scatter(g, b)

                    @pl.when(g + 4 < n_my)
                    def _():
                        pltpu.async_copy(ep_hbm.at[wid + (g + 4) * _NW],
                                         idx[(b + 4) % 6], isem[(b + 4) % 6])
            return 0

        lax.fori_loop(0, (n_my + 5) // 6, outer, 0)
        for k in range(3):
            @pl.when(n_my % 3 == k)
            def _(k=k):
                for back in (2, 1):
                    pltpu.make_async_copy(
                        rows[(k + 3 - back) % 3],
                        agg_sp.at[idx[0].at[1, 0]],
                        ssem[(k + 3 - back) % 3]).wait()
        plsc.subcore_barrier()
        pltpu.sync_copy(agg_sp.at[pl.ds(row0, rpt)],
                        out_hbm.at[c, pl.ds(row0, rpt)])
        if rem:
            @pl.when(s == _NS - 1)
            def _():
                pltpu.sync_copy(agg_sp.at[pl.ds(rpt * _NS, rem)],
                                out_hbm.at[c, pl.ds(rpt * _NS, rem)])

    def wrapped(hs_hbm, ep_hbm, out_hbm,
                i0, i1, i2, i3, i4, i5, r0, r1, r2, agg_sp,
                g0, g1, g2, s0, s1, s2, p0, p1, p2, p3, p4, p5):
        body(hs_hbm, ep_hbm, out_hbm, (i0, i1, i2, i3, i4, i5),
             (r0, r1, r2), agg_sp, (g0, g1, g2), (s0, s1, s2),
             (p0, p1, p2, p3, p4, p5))

    ivm = pltpu.VMEM((2, _NSUB, _SUB), jnp.int32)
    rvm = pltpu.VMEM((_GC, d), jnp.float32)
    return pl.kernel(
        wrapped,
        out_type=jax.ShapeDtypeStruct((_NC, n, d), jnp.float32),
        mesh=_sc_mesh(),
        scratch_types=[
            ivm, ivm, ivm, ivm, ivm, ivm,
            rvm, rvm, rvm,
            pltpu.VMEM_SHARED((n, d), jnp.float32),
        ] + [pltpu.SemaphoreType.DMA] * 12,
    )


def _prep_call(n, d, blk):
    g = (n + blk - 1) // blk

    def body(x_ref, dp_ref, hs_ref):
        dpt = jnp.transpose(dp_ref[...])
        deg_out = dpt[:, 0:1] + dpt[:, 2:3]
        hs_ref[...] = x_ref[...] * lax.rsqrt(jnp.maximum(deg_out, 1.0))

    return pl.pallas_call(
        body,
        grid=(g,),
        in_specs=[pl.BlockSpec((blk, d), lambda i: (i, 0)),
                  pl.BlockSpec((4, blk), lambda i: (0, i))],
        out_specs=pl.BlockSpec((blk, d), lambda i: (i, 0)),
        out_shape=jax.ShapeDtypeStruct((n, d), jnp.float32),
    )


def _layer_call(n, d, blk, last):
    g = (n + blk - 1) // blk

    def body(p_ref, dp_ref, w_ref, b_ref, o_ref):
        agg = p_ref[0] + p_ref[1]
        dpt = jnp.transpose(dp_ref[...])
        deg_in = dpt[:, 1:2] + dpt[:, 3:4]
        agg = agg * lax.rsqrt(jnp.maximum(deg_in, 1.0))
        y = jnp.dot(agg, w_ref[...], preferred_element_type=jnp.float32)
        y = y + b_ref[...]
        if not last:
            deg_out = dpt[:, 0:1] + dpt[:, 2:3]
            y = y * lax.rsqrt(jnp.maximum(deg_out, 1.0))
        o_ref[...] = y

    return pl.pallas_call(
        body,
        grid=(g,),
        in_specs=[pl.BlockSpec((_NC, blk, d), lambda i: (0, i, 0)),
                  pl.BlockSpec((4, blk), lambda i: (0, i)),
                  pl.BlockSpec((d, d), lambda i: (0, 0)),
                  pl.BlockSpec((1, d), lambda i: (0, 0))],
        out_specs=pl.BlockSpec((blk, d), lambda i: (i, 0)),
        out_shape=jax.ShapeDtypeStruct((n, d), jnp.float32),
    )


def kernel(node_embeddings, edge_matrix, edge_labels_or_dummynode_ids,
           W0, b0, W1, b1, W2, b2, W3, b3):
    del edge_labels_or_dummynode_ids
    x = node_embeddings
    n, d = x.shape
    e = edge_matrix.shape[1]
    assert e % _GC == 0 and e % _DSUB == 0 and n % _NS == 0 and d % 16 == 0
    n_groups = e // _GC
    blk = 2048
    n_pad = ((n + blk - 1) // blk) * blk

    ep = edge_matrix.reshape(2, n_groups, _NSUB, _SUB).transpose(1, 0, 2, 3)
    epd = edge_matrix.reshape(2, e // _DSUB, _DSUB).transpose(1, 0, 2)

    degp = _deg_call(n_pad, e // _DSUB)(epd)
    dp = degp.reshape(2 * _NC, n_pad)

    hs = _prep_call(n, d, blk)(x, dp)
    for i, (w, b) in enumerate(((W0, b0), (W1, b1), (W2, b2), (W3, b3))):
        partial = _agg_call(n, d, n_groups)(hs, ep)
        hs = _layer_call(n, d, blk, i == 3)(partial, dp, w, b.reshape(1, d))
    return hs
